# baseline (speedup 1.0000x reference)
import functools

import jax
import jax.numpy as jnp
from jax import lax
from jax.experimental import pallas as pl
from jax.experimental.pallas import tpu as pltpu
from jax.experimental.pallas import tpu_sc as plsc

_BLK = 256
_CHUNK = 64


def _routing(topk_ids, num_experts, blk):
    flat_ids = topk_ids.reshape(-1).astype(jnp.int32)
    n = flat_ids.shape[0]
    oh = (flat_ids[:, None] == jnp.arange(num_experts, dtype=jnp.int32)[None, :])
    ohi = oh.astype(jnp.int32)
    csum = jnp.cumsum(ohi, axis=0)
    rank = jnp.sum(jnp.where(oh, csum - 1, 0), axis=1)
    counts = csum[-1]
    padded = ((counts + blk - 1) // blk) * blk
    ends = jnp.cumsum(padded)
    off = ends - padded
    dst = rank + jnp.sum(jnp.where(oh, off[None, :], 0), axis=1)
    num_blocks = (n + num_experts * blk) // blk
    b_start = jnp.arange(num_blocks, dtype=jnp.int32) * blk
    block_expert = jnp.minimum(
        jnp.sum((b_start[:, None] >= ends[None, :]).astype(jnp.int32), axis=1),
        num_experts - 1).astype(jnp.int32)
    return dst.astype(jnp.int32), block_expert


def _sc_dispatch(x, tok_of_slot, dst, np_rows):
    n, d = tok_of_slot.shape[0], x.shape[1]
    info = plsc.get_sparse_core_info()
    nw = info.num_cores * info.num_subcores
    per_w = n // nw

    mesh = plsc.VectorSubcoreMesh(core_axis_name="c", subcore_axis_name="s")

    @functools.partial(
        pl.kernel, mesh=mesh,
        out_type=jax.ShapeDtypeStruct((np_rows, d), jnp.float32),
        scratch_types=[
            pltpu.VMEM((_CHUNK,), jnp.int32),
            pltpu.VMEM((_CHUNK,), jnp.int32),
            pltpu.VMEM((_CHUNK, d), jnp.float32),
            pltpu.SemaphoreType.DMA,
        ],
    )
    def disp(x_hbm, tok_hbm, dst_hbm, xg_hbm, tok_v, dst_v, rows_v, sem):
        wid = lax.axis_index("s") * info.num_cores + lax.axis_index("c")
        for c in range(per_w // _CHUNK):
            base = wid * per_w + c * _CHUNK
            pltpu.sync_copy(tok_hbm.at[pl.ds(base, _CHUNK)], tok_v)
            pltpu.sync_copy(dst_hbm.at[pl.ds(base, _CHUNK)], dst_v)
            pltpu.async_copy(x_hbm.at[tok_v], rows_v, sem).wait()
            pltpu.async_copy(rows_v, xg_hbm.at[dst_v], sem).wait()

    return disp(x, tok_of_slot, dst)


def _sc_unpermute(y, dst, n):
    d = y.shape[1]
    info = plsc.get_sparse_core_info()
    nw = info.num_cores * info.num_subcores
    per_w = n // nw

    mesh = plsc.VectorSubcoreMesh(core_axis_name="c", subcore_axis_name="s")

    @functools.partial(
        pl.kernel, mesh=mesh,
        out_type=jax.ShapeDtypeStruct((n, d), jnp.float32),
        scratch_types=[
            pltpu.VMEM((_CHUNK,), jnp.int32),
            pltpu.VMEM((_CHUNK, d), jnp.float32),
            pltpu.SemaphoreType.DMA,
        ],
    )
    def unperm(y_hbm, dst_hbm, yf_hbm, idx_v, rows_v, sem):
        wid = lax.axis_index("s") * info.num_cores + lax.axis_index("c")
        for c in range(per_w // _CHUNK):
            base = wid * per_w + c * _CHUNK
            pltpu.sync_copy(dst_hbm.at[pl.ds(base, _CHUNK)], idx_v)
            pltpu.async_copy(y_hbm.at[idx_v], rows_v, sem).wait()
            pltpu.sync_copy(rows_v, yf_hbm.at[pl.ds(base, _CHUNK)])

    return unperm(y, dst)


def _gemm_body(dff, be_ref, xg_ref, w1_ref, w2_ref, y_ref):
    e = be_ref[pl.program_id(0)]
    x = xg_ref[...].astype(jnp.bfloat16)
    h = jax.lax.dot_general(
        x, w1_ref[e], (((1,), (1,)), ((), ())),
        preferred_element_type=jnp.float32)
    gate = h[:, :dff]
    up = h[:, dff:]
    act = (gate * jax.lax.logistic(gate) * up).astype(jnp.bfloat16)
    y_ref[...] = jax.lax.dot_general(
        act, w2_ref[e], (((1,), (1,)), ((), ())),
        preferred_element_type=jnp.float32)


def _combine_body(d, yf_ref, tw_ref, o_ref):
    yf = yf_ref[...]
    tw = tw_ref[...]
    o_ref[...] = (tw[:, 0:1] * yf[:, :d] + tw[:, 1:2] * yf[:, d:])


def kernel(hidden_states, w1, w2, topk_weights, topk_ids):
    num_tokens, d = hidden_states.shape
    num_experts = w1.shape[0]
    dff = w2.shape[2]
    k = topk_ids.shape[1]
    n = num_tokens * k
    np_rows = n + num_experts * _BLK
    num_blocks = np_rows // _BLK

    dst, block_expert = _routing(topk_ids, num_experts, _BLK)
    tok_of_slot = (jnp.arange(n, dtype=jnp.int32) // k)

    xg = _sc_dispatch(hidden_states, tok_of_slot, dst, np_rows)

    w1b = w1.astype(jnp.bfloat16)
    w2b = w2.astype(jnp.bfloat16)

    y = pl.pallas_call(
        functools.partial(_gemm_body, dff),
        grid_spec=pltpu.PrefetchScalarGridSpec(
            num_scalar_prefetch=1,
            grid=(num_blocks,),
            in_specs=[
                pl.BlockSpec((_BLK, d), lambda b, be: (b, 0)),
                pl.BlockSpec((num_experts, 2 * dff, d), lambda b, be: (0, 0, 0)),
                pl.BlockSpec((num_experts, d, dff), lambda b, be: (0, 0, 0)),
            ],
            out_specs=pl.BlockSpec((_BLK, d), lambda b, be: (b, 0)),
        ),
        out_shape=jax.ShapeDtypeStruct((np_rows, d), jnp.float32),
    )(block_expert, xg, w1b, w2b)

    yflat = _sc_unpermute(y, dst, n)

    bt = 512
    out = pl.pallas_call(
        functools.partial(_combine_body, d),
        grid=(num_tokens // bt,),
        in_specs=[
            pl.BlockSpec((bt, k * d), lambda t: (t, 0)),
            pl.BlockSpec((bt, k), lambda t: (t, 0)),
        ],
        out_specs=pl.BlockSpec((bt, d), lambda t: (t, 0)),
        out_shape=jax.ShapeDtypeStruct((num_tokens, d), jnp.float32),
    )(yflat.reshape(num_tokens, k * d), topk_weights)
    return out
